# Initial kernel scaffold; baseline (speedup 1.0000x reference)
import jax
import jax.numpy as jnp
from jax.experimental import pallas as pl


def kernel(x, edge_index, W1l, W1r, b1, W2l, W2r, b2, W3l, W3r, b3):
    raise NotImplementedError("write your pallas kernel here")


# speedup vs baseline: 1.3169x; 1.3169x over previous
import functools

import jax
import jax.numpy as jnp
from jax import lax
from jax.experimental import pallas as pl
from jax.experimental.pallas import tpu as pltpu
from jax.experimental.pallas import tpu_sc as plsc

N_NODES = 10000
N_EDGES = 160000
D_IN = 256
D_HID = 512

NC = 2
NS = 16
NW = NC * NS
EB = 128
K_TILE = 5120
NB = K_TILE // EB
E_PAD = NW * K_TILE
N_ACC = 10112
ROWS_SC = N_ACC // NS
CW = 64
CNT_W = 16
R_BLK = 1000


def _make_sc_agg(n_chunks: int, with_counts: bool):
    mesh = plsc.VectorSubcoreMesh(core_axis_name="c", subcore_axis_name="s")
    out_types = [jax.ShapeDtypeStruct((NC, n_chunks, N_ACC, CW), jnp.float32)]
    scratch = [
        pltpu.VMEM((EB,), jnp.int32),
        pltpu.VMEM((EB,), jnp.int32),
        pltpu.VMEM((EB,), jnp.int32),
        pltpu.VMEM((EB, CW), jnp.float32),
        pltpu.VMEM_SHARED((N_ACC, CW), jnp.float32),
        pltpu.SemaphoreType.DMA,
    ]
    if with_counts:
        out_types.append(jax.ShapeDtypeStruct((NC, N_ACC, CNT_W), jnp.float32))
        scratch += [
            pltpu.VMEM((EB, CNT_W), jnp.float32),
            pltpu.VMEM((ROWS_SC, CNT_W), jnp.float32),
            pltpu.VMEM_SHARED((N_ACC, CNT_W), jnp.float32),
        ]

    pieces = []
    off = 0
    while off + EB < ROWS_SC:
        pieces.append(off)
        off += EB
    pieces.append(ROWS_SC - EB)

    def body(*refs):
        if with_counts:
            (xf, srcp, dstp, zrow, zcnt, onesb,
             p_out, cnt_out,
             srcv, srcov, dstv, rows, acc, sem, onesv, cstage, cntacc) = refs
        else:
            (xf, srcp, dstp, zrow,
             p_out,
             srcv, srcov, dstv, rows, acc, sem) = refs
        cid = lax.axis_index("c")
        sid = lax.axis_index("s")
        wid = cid * NS + sid
        ebase = wid * K_TILE
        rbase = sid * ROWS_SC
        if with_counts:
            pltpu.sync_copy(onesb, onesv)
            pltpu.sync_copy(zcnt.at[pl.ds(rbase, ROWS_SC)], cstage)
            pltpu.sync_copy(cstage, cntacc.at[pl.ds(rbase, ROWS_SC)])
        for c in range(n_chunks):
            pltpu.sync_copy(zrow.at[pl.ds(0, EB)], rows)
            for po in pieces:
                pltpu.sync_copy(rows, acc.at[pl.ds(rbase + po, EB)])
            plsc.subcore_barrier()

            def blk(b, carry, c=c):
                base = ebase + b * EB
                pltpu.sync_copy(srcp.at[pl.ds(base, EB)], srcv)
                pltpu.sync_copy(dstp.at[pl.ds(base, EB)], dstv)
                if c == 0:
                    idx = srcv
                else:
                    for j in range(EB // 16):
                        srcov[pl.ds(j * 16, 16)] = (
                            srcv[pl.ds(j * 16, 16)] + c * N_NODES)
                    idx = srcov
                pltpu.async_copy(xf.at[idx], rows, sem).wait()
                pltpu.sync_copy(rows, acc.at[dstv], add=True)
                if with_counts and c == 0:
                    pltpu.sync_copy(onesv, cntacc.at[dstv], add=True)
                return carry

            lax.fori_loop(0, NB, blk, 0)
            plsc.subcore_barrier()
            for po in pieces:
                pltpu.sync_copy(acc.at[pl.ds(rbase + po, EB)], rows)
                pltpu.sync_copy(rows, p_out.at[cid, c, pl.ds(rbase + po, EB)])
            if with_counts and c == 0:
                pltpu.sync_copy(cntacc.at[pl.ds(rbase, ROWS_SC)], cstage)
                pltpu.sync_copy(cstage, cnt_out.at[cid, pl.ds(rbase, ROWS_SC)])

    out_type = tuple(out_types) if with_counts else out_types[0]
    return functools.partial(
        pl.kernel, mesh=mesh, out_type=out_type, scratch_types=scratch,
        compiler_params=pltpu.CompilerParams(use_tc_tiling_on_sc=False),
    )(body)


def _make_tc_layer(d_in: int, normalize: bool, leaky: bool,
                   out_chunked: bool):
    n_chunks = d_in // CW
    grid = (N_NODES // R_BLK,)
    n_out_chunks = D_HID // CW

    def body(p_ref, cnt_ref, x_ref, wl_ref, wr_ref, b_ref, o_ref, *oc_ref):
        a = jnp.concatenate(
            [p_ref[0, c] + p_ref[1, c] for c in range(n_chunks)], axis=1)
        accl = lax.dot(a, wl_ref[...],
                       precision=lax.Precision.HIGHEST,
                       preferred_element_type=jnp.float32)
        accr = lax.dot(x_ref[...], wr_ref[...],
                       precision=lax.Precision.HIGHEST,
                       preferred_element_type=jnp.float32)
        cnt = cnt_ref[0, :, 0:1] + cnt_ref[1, :, 0:1]
        inv = 1.0 / jnp.maximum(cnt, 1.0)
        out = accl * inv + accr + b_ref[...]
        if normalize:
            nrm = jnp.sqrt(jnp.sum(out * out, axis=1, keepdims=True))
            out = out / jnp.maximum(nrm, 1e-12)
        if leaky:
            out = jnp.where(out >= 0, out, 0.1 * out)
        o_ref[...] = out
        if out_chunked:
            for c2 in range(n_out_chunks):
                oc_ref[0][c2] = out[:, c2 * CW:(c2 + 1) * CW]

    out_shapes = [jax.ShapeDtypeStruct((N_NODES, D_HID), jnp.float32)]
    out_specs = [pl.BlockSpec((R_BLK, D_HID), lambda i: (i, 0))]
    if out_chunked:
        out_shapes.append(
            jax.ShapeDtypeStruct((n_out_chunks, N_NODES, CW), jnp.float32))
        out_specs.append(
            pl.BlockSpec((n_out_chunks, R_BLK, CW), lambda i: (0, i, 0)))

    return pl.pallas_call(
        body,
        grid=grid,
        in_specs=[
            pl.BlockSpec((NC, n_chunks, R_BLK, CW), lambda i: (0, 0, i, 0)),
            pl.BlockSpec((NC, R_BLK, CNT_W), lambda i: (0, i, 0)),
            pl.BlockSpec((R_BLK, d_in), lambda i: (i, 0)),
            pl.BlockSpec((d_in, D_HID), lambda i: (0, 0)),
            pl.BlockSpec((d_in, D_HID), lambda i: (0, 0)),
            pl.BlockSpec((1, D_HID), lambda i: (0, 0)),
        ],
        out_specs=out_specs,
        out_shape=out_shapes,
    )


def kernel(x, edge_index, W1l, W1r, b1, W2l, W2r, b2, W3l, W3r, b3):
    src = edge_index[0].astype(jnp.int32)
    dst = edge_index[1].astype(jnp.int32)
    pad = E_PAD - N_EDGES
    srcp = jnp.concatenate([src, jnp.zeros((pad,), jnp.int32)])
    dstp = jnp.concatenate([dst, jnp.full((pad,), N_NODES, jnp.int32)])
    zrow = jnp.zeros((EB, CW), jnp.float32)
    zcnt = jnp.zeros((N_ACC, CNT_W), jnp.float32)
    onesb = jnp.ones((EB, CNT_W), jnp.float32)

    xc = jnp.transpose(x.reshape(N_NODES, D_IN // CW, CW), (1, 0, 2))

    agg1 = _make_sc_agg(D_IN // CW, True)
    agg_h = _make_sc_agg(D_HID // CW, False)
    lyr1 = _make_tc_layer(D_IN, True, True, True)
    lyr2 = _make_tc_layer(D_HID, True, True, True)
    lyr3 = _make_tc_layer(D_HID, False, False, False)

    p1, cntp = agg1(xc.reshape(-1, CW), srcp, dstp, zrow, zcnt, onesb)
    h1, h1c = lyr1(p1, cntp, x, W1l, W1r, b1.reshape(1, D_HID))
    p2 = agg_h(h1c.reshape(-1, CW), srcp, dstp, zrow)
    h2, h2c = lyr2(p2, cntp, h1, W2l, W2r, b2.reshape(1, D_HID))
    p3 = agg_h(h2c.reshape(-1, CW), srcp, dstp, zrow)
    (out,) = lyr3(p3, cntp, h2, W3l, W3r, b3.reshape(1, D_HID))
    return out
